# baseline (speedup 1.0000x reference)
import functools

import jax
import jax.numpy as jnp
from jax import lax
from jax.experimental import pallas as pl
from jax.experimental.pallas import tpu as pltpu
from jax.experimental.pallas import tpu_sc as plsc

FREEZE_RATIO = 0.9

R = 16384
D = 1024
T = int(R * FREEZE_RATIO)
NC = 2
NS = 16
NW = NC * NS
LANES = 16
GRP = 8

GRP_LO = (T // GRP) * GRP
NZG = T - GRP_LO

CH = 32
COPY_LO = GRP_LO + GRP
NCOPY = (R - COPY_LO) // CH
NCOPY2 = NCOPY - NW

ZBLK = 776
ZGRID = GRP_LO // ZBLK


_mesh = plsc.VectorSubcoreMesh(core_axis_name="c", subcore_axis_name="s")


@functools.partial(
    pl.kernel,
    mesh=_mesh,
    out_type=[
        jax.ShapeDtypeStruct((R, D), jnp.float32),
        jax.ShapeDtypeStruct((LANES,), jnp.int32),
    ],
    scratch_types=[
        pltpu.VMEM((CH, D), jnp.float32),
        pltpu.VMEM((CH, D), jnp.float32),
        pltpu.VMEM((GRP, D), jnp.float32),
        pltpu.VMEM((LANES,), jnp.int32),
        pltpu.SemaphoreType.DMA,
        pltpu.SemaphoreType.DMA,
    ],
)
def _tail_sc(tokens_hbm, out_hbm, cnt_hbm,
             buf_a, buf_b, buf_m, cnt_v, sem_i, sem_o):
    wid = lax.axis_index("s") * NC + lax.axis_index("c")

    copy_a = COPY_LO + wid * CH
    in_a = pltpu.async_copy(tokens_hbm.at[pl.ds(copy_a, CH)], buf_a, sem_i)

    copy_b = COPY_LO + (NW + wid) * CH

    @pl.when(wid < NCOPY2)
    def _fire_in_b():
        pltpu.async_copy(tokens_hbm.at[pl.ds(copy_b, CH)], buf_b, sem_i)

    @pl.when(wid == NW - 1)
    def _fire_in_m():
        pltpu.async_copy(tokens_hbm.at[pl.ds(GRP_LO, GRP)], buf_m, sem_i)

    in_a.wait()
    out_a = pltpu.async_copy(buf_a, out_hbm.at[pl.ds(copy_a, CH)], sem_o)

    @pl.when(wid < NCOPY2)
    def _flush_b():
        pltpu.make_async_copy(tokens_hbm.at[pl.ds(copy_b, CH)],
                              buf_b, sem_i).wait()
        pltpu.async_copy(buf_b, out_hbm.at[pl.ds(copy_b, CH)], sem_o).wait()

    @pl.when(wid == NW - 1)
    def _flush_m():
        pltpu.make_async_copy(tokens_hbm.at[pl.ds(GRP_LO, GRP)],
                              buf_m, sem_i).wait()
        for r in range(NZG):
            for c in range(D // LANES):
                buf_m[r, pl.ds(c * LANES, LANES)] = jnp.zeros(
                    (LANES,), jnp.float32)
        pltpu.async_copy(buf_m, out_hbm.at[pl.ds(GRP_LO, GRP)], sem_o).wait()

    @pl.when(wid == 0)
    def _write_count():
        cnt_v[...] = jnp.full((LANES,), T, jnp.int32)
        pltpu.sync_copy(cnt_v, cnt_hbm)

    out_a.wait()


def _zero_prefix_body(x_hbm, o_ref):
    del x_hbm
    o_ref[...] = jnp.zeros_like(o_ref)


_zero_prefix = pl.pallas_call(
    _zero_prefix_body,
    grid=(ZGRID,),
    in_specs=[pl.BlockSpec(memory_space=pl.ANY)],
    out_specs=pl.BlockSpec((ZBLK, D), lambda i: (i, 0)),
    out_shape=jax.ShapeDtypeStruct((R, D), jnp.float32),
    input_output_aliases={0: 0},
)


@jax.jit
def kernel(tokens):
    b, s, d = tokens.shape
    flat = tokens.reshape(b * s, d)
    tail, cnt = _tail_sc(flat)
    out_flat = _zero_prefix(tail)
    return out_flat.reshape(b, s, d), cnt[0]
